# baseline (speedup 1.0000x reference)
import functools

import jax
import jax.numpy as jnp
from jax import lax
from jax.experimental import pallas as pl
from jax.experimental.pallas import tpu as pltpu
from jax.experimental.pallas import tpu_sc as plsc

N_TOKENS = 8192
N_EXPERTS = 16
N_FEATURES = 128
LANES = 16
N_WORKERS = 32
TPW = N_TOKENS // N_WORKERS
GROUPS = TPW // LANES


def _logits_tc(u, W, b8):
    def body(u_ref, w_ref, b_ref, o_ref):
        acc = lax.dot_general(
            u_ref[...], w_ref[...],
            dimension_numbers=(((1,), (1,)), ((), ())),
            preferred_element_type=jnp.float32,
        )
        o_ref[...] = acc + b_ref[...]

    blk = 2048
    return pl.pallas_call(
        body,
        grid=(N_TOKENS // blk,),
        in_specs=[
            pl.BlockSpec((blk, N_FEATURES), lambda i: (i, 0)),
            pl.BlockSpec((N_EXPERTS, N_FEATURES), lambda i: (0, 0)),
            pl.BlockSpec((1, N_EXPERTS), lambda i: (0, 0)),
        ],
        out_specs=pl.BlockSpec((blk, N_EXPERTS), lambda i: (i, 0)),
        out_shape=jax.ShapeDtypeStruct((N_TOKENS, N_EXPERTS), jnp.float32),
    )(u, W, b8)


def _route_sc(logits):
    mesh = plsc.VectorSubcoreMesh(core_axis_name="c", subcore_axis_name="s")

    n_chunks = 4
    rows_pc = TPW // n_chunks
    groups_pc = rows_pc // LANES

    @functools.partial(
        pl.kernel,
        mesh=mesh,
        out_type=jax.ShapeDtypeStruct((N_TOKENS, N_EXPERTS), jnp.float32),
        scratch_types=[
            pltpu.VMEM((TPW, N_EXPERTS), jnp.float32),
            pltpu.VMEM((TPW, N_EXPERTS), jnp.float32),
        ] + [pltpu.SemaphoreType.DMA] * (2 * n_chunks),
        compiler_params=pltpu.CompilerParams(needs_layout_passes=False),
    )
    def k(lg_hbm, out_hbm, lg_v, out_v, *sems):
        in_sems, out_sems = sems[:n_chunks], sems[n_chunks:]
        wid = lax.axis_index("s") * 2 + lax.axis_index("c")
        base = wid * TPW

        in_copies = [
            pltpu.make_async_copy(
                lg_hbm.at[pl.ds(base + c * rows_pc, rows_pc)],
                lg_v.at[pl.ds(c * rows_pc, rows_pc)],
                in_sems[c],
            )
            for c in range(n_chunks)
        ]
        for cp in in_copies:
            cp.start()

        iota = lax.broadcasted_iota(jnp.int32, (LANES,), 0)
        neg_inf = jnp.full((LANES,), -jnp.inf, jnp.float32)

        def group(g, carry):
            rows = [g * LANES + t for t in range(LANES)]
            ls = [lg_v[r, :] for r in rows]
            outs = []
            for l in ls:
                m1 = jnp.max(l)
                i1 = plsc.all_reduce_ffs(l == m1)
                l2 = jnp.where(iota == i1, neg_inf, l)
                m2 = jnp.max(l2)
                i2 = plsc.all_reduce_ffs(l2 == m2)
                d = lax.broadcast(m2 - m1, (LANES,))
                p1 = 1.0 / (1.0 + jnp.exp(d))
                p2 = 1.0 - p1
                outs.append(jnp.where(
                    iota == i1, p1, jnp.where(iota == i2, p2, 0.0)))
            for r, o in zip(rows, outs):
                out_v[r, :] = o
            return carry

        out_copies = []
        for c in range(n_chunks):
            in_copies[c].wait()
            lax.fori_loop(c * groups_pc, (c + 1) * groups_pc, group, 0)
            cp = pltpu.make_async_copy(
                out_v.at[pl.ds(c * rows_pc, rows_pc)],
                out_hbm.at[pl.ds(base + c * rows_pc, rows_pc)],
                out_sems[c],
            )
            cp.start()
            out_copies.append(cp)
        for cp in out_copies:
            cp.wait()

    return k(logits)


def kernel(h, u, W, b):
    del h
    logits = _logits_tc(u, W, b.reshape(1, N_EXPERTS))
    return _route_sc(logits)


# speedup vs baseline: 1.0048x; 1.0048x over previous
import functools

import jax
import jax.numpy as jnp
from jax import lax
from jax.experimental import pallas as pl
from jax.experimental.pallas import tpu as pltpu
from jax.experimental.pallas import tpu_sc as plsc

N_TOKENS = 8192
N_EXPERTS = 16
N_FEATURES = 128
LANES = 16
N_WORKERS = 32
TPW = N_TOKENS // N_WORKERS
GROUPS = TPW // LANES


def _logits_tc(u, W, b8):
    def body(u_ref, w_ref, b_ref, o_ref):
        acc = lax.dot_general(
            u_ref[...], w_ref[...],
            dimension_numbers=(((1,), (1,)), ((), ())),
            preferred_element_type=jnp.float32,
        )
        o_ref[...] = acc + b_ref[...]

    return pl.pallas_call(
        body,
        out_shape=jax.ShapeDtypeStruct((N_TOKENS, N_EXPERTS), jnp.float32),
    )(u, W, b8)


def _route_sc(logits):
    mesh = plsc.VectorSubcoreMesh(core_axis_name="c", subcore_axis_name="s")

    n_chunks = 4
    rows_pc = TPW // n_chunks
    groups_pc = rows_pc // LANES

    @functools.partial(
        pl.kernel,
        mesh=mesh,
        out_type=jax.ShapeDtypeStruct((N_TOKENS, N_EXPERTS), jnp.float32),
        scratch_types=[
            pltpu.VMEM((TPW, N_EXPERTS), jnp.float32),
            pltpu.VMEM((TPW, N_EXPERTS), jnp.float32),
        ] + [pltpu.SemaphoreType.DMA] * 2,
        compiler_params=pltpu.CompilerParams(needs_layout_passes=False),
    )
    def k(lg_hbm, out_hbm, lg_v, out_v, in_sem, out_sem):
        wid = lax.axis_index("s") * 2 + lax.axis_index("c")
        base = wid * TPW

        in_copies = [
            pltpu.make_async_copy(
                lg_hbm.at[pl.ds(base + c * rows_pc, rows_pc)],
                lg_v.at[pl.ds(c * rows_pc, rows_pc)],
                in_sem,
            )
            for c in range(n_chunks)
        ]
        for cp in in_copies:
            cp.start()

        iota = lax.broadcasted_iota(jnp.int32, (LANES,), 0)
        neg_inf = jnp.full((LANES,), -jnp.inf, jnp.float32)

        def group(g, carry):
            rows = [g * LANES + t for t in range(LANES)]
            ls = [lg_v[r, :] for r in rows]
            outs = []
            for l in ls:
                m1 = jnp.max(l)
                i1 = plsc.all_reduce_ffs(l == m1)
                l2 = jnp.where(iota == i1, neg_inf, l)
                m2 = jnp.max(l2)
                i2 = plsc.all_reduce_ffs(l2 == m2)
                d = lax.broadcast(m2 - m1, (LANES,))
                p1 = 1.0 / (1.0 + jnp.exp(d))
                p2 = 1.0 - p1
                outs.append(jnp.where(
                    iota == i1, p1, jnp.where(iota == i2, p2, 0.0)))
            for r, o in zip(rows, outs):
                out_v[r, :] = o
            return carry

        out_copies = []
        for c in range(n_chunks):
            in_copies[c].wait()
            lax.fori_loop(c * groups_pc, (c + 1) * groups_pc, group, 0)
            cp = pltpu.make_async_copy(
                out_v.at[pl.ds(c * rows_pc, rows_pc)],
                out_hbm.at[pl.ds(base + c * rows_pc, rows_pc)],
                out_sem,
            )
            cp.start()
            out_copies.append(cp)
        for cp in out_copies:
            cp.wait()

    return k(logits)


def kernel(h, u, W, b):
    del h
    logits = _logits_tc(u, W, b.reshape(1, N_EXPERTS))
    return _route_sc(logits)


# speedup vs baseline: 1.0049x; 1.0001x over previous
import functools

import jax
import jax.numpy as jnp
from jax import lax
from jax.experimental import pallas as pl
from jax.experimental.pallas import tpu as pltpu
from jax.experimental.pallas import tpu_sc as plsc

N_TOKENS = 8192
N_EXPERTS = 16
LANES = 16
N_WORKERS = 32
TPW = N_TOKENS // N_WORKERS


def _logits_tc(u, W, b1):
    def body(u_ref, w_ref, b_ref, o_ref):
        acc = lax.dot_general(
            u_ref[...], w_ref[...],
            dimension_numbers=(((1,), (1,)), ((), ())),
            preferred_element_type=jnp.float32,
        )
        o_ref[...] = acc + b_ref[...]

    return pl.pallas_call(
        body,
        out_shape=jax.ShapeDtypeStruct((N_TOKENS, N_EXPERTS), jnp.float32),
    )(u, W, b1)


def _route_sc(logits):
    mesh = plsc.VectorSubcoreMesh(core_axis_name="c", subcore_axis_name="s")

    n_chunks = 4
    rows_pc = TPW // n_chunks
    groups_pc = rows_pc // LANES

    @functools.partial(
        pl.kernel,
        mesh=mesh,
        out_type=jax.ShapeDtypeStruct((N_TOKENS, N_EXPERTS), jnp.float32),
        scratch_types=[
            pltpu.VMEM((TPW, N_EXPERTS), jnp.float32),
            pltpu.VMEM((TPW, N_EXPERTS), jnp.float32),
        ] + [pltpu.SemaphoreType.DMA] * 2,
        compiler_params=pltpu.CompilerParams(needs_layout_passes=False),
    )
    def k(lg_hbm, out_hbm, lg_v, out_v, in_sem, out_sem):
        wid = lax.axis_index("s") * 2 + lax.axis_index("c")
        base = wid * TPW

        in_copies = [
            pltpu.make_async_copy(
                lg_hbm.at[pl.ds(base + c * rows_pc, rows_pc)],
                lg_v.at[pl.ds(c * rows_pc, rows_pc)],
                in_sem,
            )
            for c in range(n_chunks)
        ]
        for cp in in_copies:
            cp.start()

        iota = lax.broadcasted_iota(jnp.int32, (LANES,), 0)
        neg_inf = jnp.full((LANES,), -jnp.inf, jnp.float32)

        def group(g, carry):
            rows = [g * LANES + t for t in range(LANES)]
            ls = [lg_v[r, :] for r in rows]
            outs = []
            for l in ls:
                m1 = jnp.max(l)
                i1 = plsc.all_reduce_ffs(l == m1)
                l2 = jnp.where(iota == i1, neg_inf, l)
                m2 = jnp.max(l2)
                i2 = plsc.all_reduce_ffs(l2 == m2)
                d = lax.broadcast(m2 - m1, (LANES,))
                p1 = 1.0 / (1.0 + jnp.exp(d))
                p2 = 1.0 - p1
                outs.append(jnp.where(
                    iota == i1, p1, jnp.where(iota == i2, p2, 0.0)))
            for r, o in zip(rows, outs):
                out_v[r, :] = o
            return carry

        out_copies = []
        for c in range(n_chunks):
            in_copies[c].wait()
            lax.fori_loop(c * groups_pc, (c + 1) * groups_pc, group, 0)
            cp = pltpu.make_async_copy(
                out_v.at[pl.ds(c * rows_pc, rows_pc)],
                out_hbm.at[pl.ds(base + c * rows_pc, rows_pc)],
                out_sem,
            )
            cp.start()
            out_copies.append(cp)
        for cp in out_copies:
            cp.wait()

    return k(logits)


def kernel(h, u, W, b):
    del h
    logits = _logits_tc(u, W, b.reshape(1, N_EXPERTS))
    return _route_sc(logits)


# speedup vs baseline: 1.0468x; 1.0417x over previous
import functools

import jax
import jax.numpy as jnp
from jax import lax
from jax.experimental import pallas as pl
from jax.experimental.pallas import tpu as pltpu
from jax.experimental.pallas import tpu_sc as plsc

N_TOKENS = 8192
N_EXPERTS = 16
LANES = 16
N_WORKERS = 32
TPW = N_TOKENS // N_WORKERS


def _logits_tc(u, W, b1):
    def body(u_ref, w_ref, b_ref, o_ref):
        acc = lax.dot_general(
            u_ref[...], w_ref[...],
            dimension_numbers=(((1,), (1,)), ((), ())),
            preferred_element_type=jnp.float32,
        )
        o_ref[...] = acc + b_ref[...]

    return pl.pallas_call(
        body,
        out_shape=jax.ShapeDtypeStruct((N_TOKENS, N_EXPERTS), jnp.float32),
    )(u, W, b1)


def _route_sc(logits):
    mesh = plsc.VectorSubcoreMesh(core_axis_name="c", subcore_axis_name="s")

    @functools.partial(
        pl.kernel,
        mesh=mesh,
        out_type=jax.ShapeDtypeStruct((N_TOKENS, N_EXPERTS), jnp.float32),
        scratch_types=[
            pltpu.VMEM((TPW, N_EXPERTS), jnp.float32),
            pltpu.VMEM((TPW, N_EXPERTS), jnp.float32),
        ],
        compiler_params=pltpu.CompilerParams(needs_layout_passes=False),
    )
    def k(lg_hbm, out_hbm, lg_v, out_v):
        wid = lax.axis_index("s") * 2 + lax.axis_index("c")
        base = wid * TPW
        pltpu.sync_copy(lg_hbm.at[pl.ds(base, TPW)], lg_v)

        iota = lax.broadcasted_iota(jnp.int32, (LANES,), 0)
        neg_inf = jnp.full((LANES,), -jnp.inf, jnp.float32)

        def token(t, carry):
            l = lg_v[t, :]
            m1 = jnp.max(l)
            i1 = plsc.all_reduce_ffs(l == m1)
            l2 = jnp.where(iota == i1, neg_inf, l)
            m2 = jnp.max(l2)
            i2 = plsc.all_reduce_ffs(l2 == m2)
            d = lax.broadcast(m2 - m1, (LANES,))
            p1 = 1.0 / (1.0 + jnp.exp(d))
            p2 = 1.0 - p1
            out_v[t, :] = jnp.where(
                iota == i1, p1, jnp.where(iota == i2, p2, 0.0))
            return carry

        lax.fori_loop(0, TPW, token, 0)
        pltpu.sync_copy(out_v, out_hbm.at[pl.ds(base, TPW)])

    return k(logits)


def kernel(h, u, W, b):
    del h
    logits = _logits_tc(u, W, b.reshape(1, N_EXPERTS))
    return _route_sc(logits)
